# Initial kernel scaffold; baseline (speedup 1.0000x reference)
import jax
import jax.numpy as jnp
from jax.experimental import pallas as pl


def kernel(x, edge_index, edge_label_index, w1, b1, w2, b2, eps0, ln0_g, ln0_b, w3, b3, w4, b4, eps1, ln1_g, ln1_b, we1, be1, we2, be2, we3, be3):
    raise NotImplementedError("write your pallas kernel here")


# speedup vs baseline: 2.5486x; 2.5486x over previous
import functools

import jax
import jax.numpy as jnp
from jax import lax
from jax.experimental import pallas as pl
from jax.experimental.pallas import tpu as pltpu
from jax.experimental.pallas import tpu_sc as plsc

N = 50000
E = 800000
EL = 100000
H = 64

HALF = N // 2
HALF_P = 25600
K = 80
EPT = E // 16
N_CHUNKS = EPT // K
OC = 40
N_OUT = HALF // OC

_MESH = dict(core_axis_name="c", subcore_axis_name="s", num_cores=2,
             num_subcores=16)


def _make_segsum(D):
  def body(tab, src, dst, out, agg_sh, src_v, dst_v, ldst_v, rows_v, obuf_v,
           sem):
    c = lax.axis_index("c")
    s = lax.axis_index("s")
    base = c * HALF

    for r in range(K):
      for j in range(D // 16):
        rows_v[r, pl.ds(j * 16, 16)] = jnp.zeros((16,), jnp.float32)
    stripe = HALF_P // 16
    for t in range(stripe // K):
      off = pl.multiple_of(s * stripe + t * K, 8)
      pltpu.sync_copy(rows_v, agg_sh.at[pl.ds(off, K), :])
    plsc.subcore_barrier()

    def chunk(t, carry):
      off = pl.multiple_of(s * EPT + t * K, 8)
      pltpu.sync_copy(src.at[pl.ds(off, K)], src_v)
      pltpu.sync_copy(dst.at[pl.ds(off, K)], dst_v)
      for j in range(K // 16):
        d = dst_v[pl.ds(j * 16, 16)]
        l = d - base
        valid = (l >= 0) & (l < HALF)
        ldst_v[pl.ds(j * 16, 16)] = jnp.where(valid, l, HALF)
      pltpu.async_copy(tab.at[src_v], rows_v, sem).wait()
      pltpu.sync_copy(rows_v, agg_sh.at[ldst_v], add=True)
      return carry

    lax.fori_loop(0, N_CHUNKS, chunk, 0)
    plsc.subcore_barrier()

    def ochunk(m, carry):
      cid = m * 16 + s

      @pl.when(cid < N_OUT)
      def _():
        off = pl.multiple_of(cid * OC, 8)
        pltpu.sync_copy(agg_sh.at[pl.ds(off, OC), :], obuf_v)
        goff = pl.multiple_of(base + cid * OC, 8)
        pltpu.sync_copy(obuf_v, out.at[pl.ds(goff, OC), :])
      return carry

    lax.fori_loop(0, (N_OUT + 15) // 16, ochunk, 0)

  return pl.kernel(
      body,
      out_type=jax.ShapeDtypeStruct((N, D), jnp.float32),
      mesh=plsc.VectorSubcoreMesh(**_MESH),
      compiler_params=pltpu.CompilerParams(use_tc_tiling_on_sc=False),
      scratch_types=[
          pltpu.VMEM_SHARED((HALF_P, D), jnp.float32),
          pltpu.VMEM((K,), jnp.int32),
          pltpu.VMEM((K,), jnp.int32),
          pltpu.VMEM((K,), jnp.int32),
          pltpu.VMEM((K, D), jnp.float32),
          pltpu.VMEM((OC, D), jnp.float32),
          pltpu.SemaphoreType.DMA,
      ],
  )


_segsum16 = _make_segsum(16)
_segsum64 = _make_segsum(H)

_GCH = EL // K


def _pair_gather_body(tab, sidx, didx, hs, hd, si_v, rows_v, sem):
  c = lax.axis_index("c")
  s = lax.axis_index("s")
  wid = s * 2 + c

  def chunk(m, carry):
    cid = m * 32 + wid

    @pl.when(cid < _GCH)
    def _():
      off = pl.multiple_of(cid * K, 8)
      pltpu.sync_copy(sidx.at[pl.ds(off, K)], si_v)
      pltpu.async_copy(tab.at[si_v], rows_v, sem).wait()
      pltpu.sync_copy(rows_v, hs.at[pl.ds(off, K), :])
      pltpu.sync_copy(didx.at[pl.ds(off, K)], si_v)
      pltpu.async_copy(tab.at[si_v], rows_v, sem).wait()
      pltpu.sync_copy(rows_v, hd.at[pl.ds(off, K), :])
    return carry

  lax.fori_loop(0, (_GCH + 31) // 32, chunk, 0)


_pair_gather = pl.kernel(
    _pair_gather_body,
    out_type=(jax.ShapeDtypeStruct((EL, H), jnp.float32),
              jax.ShapeDtypeStruct((EL, H), jnp.float32)),
    mesh=plsc.VectorSubcoreMesh(**_MESH),
    compiler_params=pltpu.CompilerParams(use_tc_tiling_on_sc=False),
    scratch_types=[
        pltpu.VMEM((K,), jnp.int32),
        pltpu.VMEM((K, H), jnp.float32),
        pltpu.SemaphoreType.DMA,
    ],
)

BM = 1000


def _dense_body(x_ref, agg_ref, w1_ref, b1_ref, w2_ref, b2_ref, eps_ref,
                g_ref, b_ref, out_ref):
  eps = eps_ref[0, 0]
  u = (1.0 + eps) * x_ref[...] + agg_ref[...]
  pre = jnp.maximum(
      jnp.dot(u, w1_ref[...], preferred_element_type=jnp.float32)
      + b1_ref[...], 0.0)
  h = jnp.dot(pre, w2_ref[...], preferred_element_type=jnp.float32) + b2_ref[...]
  mu = jnp.mean(h, axis=1, keepdims=True)
  var = jnp.mean((h - mu) ** 2, axis=1, keepdims=True)
  out_ref[...] = (h - mu) / jnp.sqrt(var + 1e-5) * g_ref[...] + b_ref[...]


def _make_dense(din):
  full = lambda shp: pl.BlockSpec(shp, lambda i: (0, 0))
  return pl.pallas_call(
      _dense_body,
      grid=(N // BM,),
      in_specs=[
          pl.BlockSpec((BM, din), lambda i: (i, 0)),
          pl.BlockSpec((BM, din), lambda i: (i, 0)),
          full((din, H)),
          full((1, H)),
          full((H, H)),
          full((1, H)),
          full((1, 1)),
          full((1, H)),
          full((1, H)),
      ],
      out_specs=pl.BlockSpec((BM, H), lambda i: (i, 0)),
      out_shape=jax.ShapeDtypeStruct((N, H), jnp.float32),
  )


_dense16 = _make_dense(16)
_dense64 = _make_dense(H)


def _head_body(hs_ref, hd_ref, wa_ref, wb_ref, b1_ref, w2_ref, b2_ref,
               w3_ref, b3_ref, out_ref):
  z = jnp.maximum(
      jnp.dot(hs_ref[...], wa_ref[...], preferred_element_type=jnp.float32)
      + jnp.dot(hd_ref[...], wb_ref[...], preferred_element_type=jnp.float32)
      + b1_ref[...], 0.0)
  z = jnp.maximum(
      jnp.dot(z, w2_ref[...], preferred_element_type=jnp.float32)
      + b2_ref[...], 0.0)
  w3row = w3_ref[0:1, :]
  out_ref[...] = jnp.sum(z * w3row, axis=1, keepdims=True) + b3_ref[...]


_head = pl.pallas_call(
    _head_body,
    grid=(EL // BM,),
    in_specs=[
        pl.BlockSpec((BM, H), lambda i: (i, 0)),
        pl.BlockSpec((BM, H), lambda i: (i, 0)),
        pl.BlockSpec((H, H), lambda i: (0, 0)),
        pl.BlockSpec((H, H), lambda i: (0, 0)),
        pl.BlockSpec((1, H), lambda i: (0, 0)),
        pl.BlockSpec((H, H // 2), lambda i: (0, 0)),
        pl.BlockSpec((1, H // 2), lambda i: (0, 0)),
        pl.BlockSpec((8, H // 2), lambda i: (0, 0)),
        pl.BlockSpec((1, 1), lambda i: (0, 0)),
    ],
    out_specs=pl.BlockSpec((BM, 1), lambda i: (i, 0)),
    out_shape=jax.ShapeDtypeStruct((EL, 1), jnp.float32),
)


def kernel(x, edge_index, edge_label_index, w1, b1, w2, b2, eps0, ln0_g,
           ln0_b, w3, b3, w4, b4, eps1, ln1_g, ln1_b, we1, be1, we2, be2,
           we3, be3):
  src = edge_index[0]
  dst = edge_index[1]
  xp = jnp.pad(x, ((0, 0), (0, 14)))
  w1p = jnp.pad(w1, ((0, 14), (0, 0)))

  agg0 = _segsum16(xp, src, dst)
  h0 = _dense16(xp, agg0, w1p, b1.reshape(1, H), w2, b2.reshape(1, H),
                eps0.reshape(1, 1), ln0_g.reshape(1, H), ln0_b.reshape(1, H))
  agg1 = _segsum64(h0, src, dst)
  h1 = _dense64(h0, agg1, w3, b3.reshape(1, H), w4, b4.reshape(1, H),
                eps1.reshape(1, 1), ln1_g.reshape(1, H), ln1_b.reshape(1, H))
  hs, hd = _pair_gather(h1, edge_label_index[0], edge_label_index[1])
  w3p = jnp.pad(we3.T, ((0, 7), (0, 0)))
  logits = _head(hs, hd, we1[:H], we1[H:], be1.reshape(1, H), we2,
                 be2.reshape(1, H // 2), w3p, be3.reshape(1, 1))
  return logits[:, 0]


# speedup vs baseline: 4.1066x; 1.6113x over previous
import functools

import jax
import jax.numpy as jnp
from jax import lax
from jax.experimental import pallas as pl
from jax.experimental.pallas import tpu as pltpu
from jax.experimental.pallas import tpu_sc as plsc

N = 50000
E = 800000
EL = 100000
H = 64

HALF = N // 2
HALF_P = 25040
ZC = 40
N_Z = HALF_P // ZC
K = 128
CPB = 8
IB = K * CPB
NBLK = 50
NPAIR = NBLK // 2
NCH = NBLK * CPB
EPT_E = NCH * K
E_PAD = 16 * EPT_E
OC = 40
N_OUT = HALF // OC

_MESH = dict(core_axis_name="c", subcore_axis_name="s", num_cores=2,
             num_subcores=16)


def _make_segsum(D):
  def body(tab, src, dst, out, agg_sh, sblk0, sblk1, dblk0, dblk1,
           ldst0, ldst1, rows0, rows1, zbuf, obuf, gsem, ssem, isem0, isem1):
    c = lax.axis_index("c")
    s = lax.axis_index("s")
    base = c * HALF
    ebase = s * EPT_E
    sblk = (sblk0, sblk1)
    dblk = (dblk0, dblk1)
    ldst = (ldst0, ldst1)
    rows = (rows0, rows1)
    isem = (isem0, isem1)

    def zrow(r, carry):
      for j in range(D // 16):
        zbuf[r, pl.ds(j * 16, 16)] = jnp.zeros((16,), jnp.float32)
      return carry

    lax.fori_loop(0, ZC, zrow, 0)
    for m in range(40):
      cid = m * 16 + s

      @pl.when(cid < N_Z)
      def _():
        off = pl.multiple_of(cid * ZC, 8)
        pltpu.sync_copy(zbuf, agg_sh.at[pl.ds(off, ZC), :])
    plsc.subcore_barrier()

    def issue_idx(bb, blk):
      off = pl.multiple_of(ebase + blk * IB, 8)
      pltpu.async_copy(src.at[pl.ds(off, IB)], sblk[bb], isem[bb])
      pltpu.async_copy(dst.at[pl.ds(off, IB)], dblk[bb], isem[bb])

    def wait_idx(bb):
      pltpu.make_async_copy(src.at[pl.ds(0, IB)], sblk[bb], isem[bb]).wait()
      pltpu.make_async_copy(dst.at[pl.ds(0, IB)], dblk[bb], isem[bb]).wait()

    def transform(bb, kk, rb):
      for j in range(K // 16):
        d = dblk[bb][pl.ds(kk * K + j * 16, 16)]
        l = d - base
        valid = (l >= 0) & (l < HALF)
        ldst[rb][pl.ds(j * 16, 16)] = jnp.where(valid, l, HALF)

    def issue_gather(bb, kk, rb):
      pltpu.async_copy(tab.at[sblk[bb].at[pl.ds(kk * K, K)]], rows[rb], gsem)

    def wait_gather(rb):
      pltpu.make_async_copy(tab.at[sblk[0].at[pl.ds(0, K)]], rows[rb],
                            gsem).wait()

    def issue_scatter(rb):
      pltpu.async_copy(rows[rb], agg_sh.at[ldst[rb]], ssem, add=True)

    def wait_scatter(rb):
      pltpu.make_async_copy(rows[rb], agg_sh.at[ldst[rb]], ssem).wait()

    issue_idx(0, 0)
    issue_idx(1, 1)
    wait_idx(0)
    transform(0, 0, 0)
    issue_gather(0, 0, 0)

    def pair_body(i, carry):
      for k in range(16):
        bb = 0 if k < 8 else 1
        kk = k % 8
        rb = k % 2
        nrb = 1 - rb
        if k == 0:
          @pl.when(i >= 1)
          def _():
            wait_scatter(nrb)
        else:
          wait_scatter(nrb)
        wait_gather(rb)
        issue_scatter(rb)
        if kk == 7:
          nblk = 2 * i + 2 + bb

          @pl.when(nblk < NBLK)
          def _():
            issue_idx(bb, nblk)
        if k == 15:
          @pl.when(i < NPAIR - 1)
          def _():
            wait_idx(0)
            transform(0, 0, nrb)
            issue_gather(0, 0, nrb)
        elif k == 7:
          wait_idx(1)
          transform(1, 0, nrb)
          issue_gather(1, 0, nrb)
        else:
          b2 = 0 if k < 7 else 1
          transform(b2, kk + 1, nrb)
          issue_gather(b2, kk + 1, nrb)
      return carry

    lax.fori_loop(0, NPAIR, pair_body, 0)
    wait_scatter(1)
    plsc.subcore_barrier()

    for m in range(40):
      cid = m * 16 + s

      @pl.when(cid < N_OUT)
      def _():
        off = pl.multiple_of(cid * OC, 8)
        pltpu.sync_copy(agg_sh.at[pl.ds(off, OC), :], obuf)
        goff = pl.multiple_of(base + cid * OC, 8)
        pltpu.sync_copy(obuf, out.at[pl.ds(goff, OC), :])

  return pl.kernel(
      body,
      out_type=jax.ShapeDtypeStruct((N, D), jnp.float32),
      mesh=plsc.VectorSubcoreMesh(**_MESH),
      compiler_params=pltpu.CompilerParams(use_tc_tiling_on_sc=False),
      scratch_types=[
          pltpu.VMEM_SHARED((HALF_P, D), jnp.float32),
          pltpu.VMEM((IB,), jnp.int32),
          pltpu.VMEM((IB,), jnp.int32),
          pltpu.VMEM((IB,), jnp.int32),
          pltpu.VMEM((IB,), jnp.int32),
          pltpu.VMEM((K,), jnp.int32),
          pltpu.VMEM((K,), jnp.int32),
          pltpu.VMEM((K, D), jnp.float32),
          pltpu.VMEM((K, D), jnp.float32),
          pltpu.VMEM((ZC, D), jnp.float32),
          pltpu.VMEM((OC, D), jnp.float32),
          pltpu.SemaphoreType.DMA,
          pltpu.SemaphoreType.DMA,
          pltpu.SemaphoreType.DMA,
          pltpu.SemaphoreType.DMA,
      ],
  )


_segsum64 = _make_segsum(H)

A0_R = 3200
A0_RC = A0_R // 128
A0_BLK = 1024
A0_NB = 50
EPT0 = A0_NB * A0_BLK


def _agg0_body(xT, src, dst, out, agg_sh, xcol, acc, sblk0, sblk1, dblk0,
               dblk1, idxall, zbuf, isem0, isem1, ssem):
  c = lax.axis_index("c")
  s = lax.axis_index("s")
  sblk = (sblk0, sblk1)
  dblk = (dblk0, dblk1)
  isem = (isem0, isem1)
  ebase = s * EPT0

  pltpu.async_copy(src.at[pl.ds(pl.multiple_of(ebase, 8), A0_BLK)], sblk0,
                   isem0)
  pltpu.async_copy(dst.at[pl.ds(pl.multiple_of(ebase, 8), A0_BLK)], dblk0,
                   isem0)
  pltpu.sync_copy(xT.at[c], xcol)

  def zacc(r, carry):
    acc[r, pl.ds(0, 16)] = jnp.zeros((16,), jnp.float32)
    return carry

  lax.fori_loop(0, A0_R, zacc, 0)

  def zrow(r, carry):
    zbuf[r, pl.ds(0, 16)] = jnp.zeros((16,), jnp.float32)
    return carry

  lax.fori_loop(0, A0_R // 16, zrow, 0)
  pltpu.sync_copy(zbuf, agg_sh.at[pl.ds(s * (A0_R // 16), A0_R // 16), :])

  lane = lax.iota(jnp.int32, 16)
  for r in range(A0_RC):
    for j in range(8):
      idxall[r, pl.ds(j * 16, 16)] = lane + (r * 128 + j * 16)

  def issue_blk(p, b):
    off = pl.multiple_of(ebase + b * A0_BLK, 8)
    pltpu.async_copy(src.at[pl.ds(off, A0_BLK)], sblk[p], isem[p])
    pltpu.async_copy(dst.at[pl.ds(off, A0_BLK)], dblk[p], isem[p])

  def wait_blk(p):
    pltpu.make_async_copy(src.at[pl.ds(0, A0_BLK)], sblk[p], isem[p]).wait()
    pltpu.make_async_copy(dst.at[pl.ds(0, A0_BLK)], dblk[p], isem[p]).wait()

  def process(p):
    def step(j, carry):
      s16 = sblk[p][pl.ds(j * 16, 16)]
      d16 = dblk[p][pl.ds(j * 16, 16)]
      v = plsc.load_gather(xcol, [s16])
      row = lax.shift_right_logical(d16, 4)
      col = lax.bitwise_and(d16, 15)
      plsc.addupdate_scatter(acc, [row, col], v)
      return carry

    lax.fori_loop(0, A0_BLK // 16, step, 0)

  def blkpair(i, carry):
    for p in (0, 1):
      wait_blk(p)
      if p == 0:
        issue_blk(1, 2 * i + 1)
      else:
        @pl.when(i < A0_NB // 2 - 1)
        def _():
          issue_blk(0, 2 * i + 2)
      process(p)
    return carry

  lax.fori_loop(0, A0_NB // 2, blkpair, 0)

  plsc.subcore_barrier()
  for r in range(A0_RC):
    pltpu.async_copy(acc.at[pl.ds(r * 128, 128), :],
                     agg_sh.at[idxall.at[r]], ssem, add=True)
  for r in range(A0_RC):
    pltpu.make_async_copy(acc.at[pl.ds(0, 128), :],
                          agg_sh.at[idxall.at[0]], ssem).wait()
  plsc.subcore_barrier()
  pltpu.sync_copy(agg_sh.at[pl.ds(s * (A0_R // 16), A0_R // 16), :], zbuf)
  pltpu.sync_copy(zbuf, out.at[c, pl.ds(s * (A0_R // 16), A0_R // 16), :])


_agg0 = pl.kernel(
    _agg0_body,
    out_type=jax.ShapeDtypeStruct((2, A0_R, 16), jnp.float32),
    mesh=plsc.VectorSubcoreMesh(**_MESH),
    compiler_params=pltpu.CompilerParams(use_tc_tiling_on_sc=False,
                                         needs_layout_passes=False),
    scratch_types=[
        pltpu.VMEM_SHARED((A0_R, 16), jnp.float32),
        pltpu.VMEM((N,), jnp.float32),
        pltpu.VMEM((A0_R, 16), jnp.float32),
        pltpu.VMEM((A0_BLK,), jnp.int32),
        pltpu.VMEM((A0_BLK,), jnp.int32),
        pltpu.VMEM((A0_BLK,), jnp.int32),
        pltpu.VMEM((A0_BLK,), jnp.int32),
        pltpu.VMEM((A0_RC, 128), jnp.int32),
        pltpu.VMEM((A0_R // 16, 16), jnp.float32),
        pltpu.SemaphoreType.DMA,
        pltpu.SemaphoreType.DMA,
        pltpu.SemaphoreType.DMA,
    ],
)

GK = 80
_GCH = EL // GK
_GSLOT = (_GCH + 31) // 32


def _pair_gather_body(tab, sidx, didx, hs, hd, ibuf0, ibuf1, rows0, rows1,
                      gsem, wsem0, wsem1, isem0, isem1):
  c = lax.axis_index("c")
  s = lax.axis_index("s")
  wid = s * 2 + c
  ibuf = (ibuf0, ibuf1)
  rows = (rows0, rows1)
  wsem = (wsem0, wsem1)
  isem = (isem0, isem1)
  idx_in = (sidx, didx)
  out = (hs, hd)

  def issue_idx(q, cid):
    off = pl.multiple_of(cid * GK, 8)
    pltpu.async_copy(idx_in[q].at[pl.ds(off, GK)], ibuf[q], isem[q])

  issue_idx(0, wid)
  issue_idx(1, wid)

  def slot(m, carry):
    cid = m * 32 + wid

    for q in (0, 1):
      @pl.when(cid < _GCH)
      def _():
        pltpu.make_async_copy(idx_in[q].at[pl.ds(0, GK)], ibuf[q],
                              isem[q]).wait()

        @pl.when(m >= 1)
        def _():
          pltpu.make_async_copy(rows[q], out[q].at[pl.ds(0, GK), :],
                                wsem[q]).wait()
        pltpu.async_copy(tab.at[ibuf[q]], rows[q], gsem).wait()
        off = pl.multiple_of(cid * GK, 8)
        pltpu.async_copy(rows[q], out[q].at[pl.ds(off, GK), :], wsem[q])

      @pl.when(cid + 32 < _GCH)
      def _():
        issue_idx(q, cid + 32)
    return carry

  lax.fori_loop(0, _GSLOT, slot, 0)
  pltpu.make_async_copy(rows0, hs.at[pl.ds(0, GK), :], wsem0).wait()
  pltpu.make_async_copy(rows1, hd.at[pl.ds(0, GK), :], wsem1).wait()


_pair_gather = pl.kernel(
    _pair_gather_body,
    out_type=(jax.ShapeDtypeStruct((EL, H), jnp.float32),
              jax.ShapeDtypeStruct((EL, H), jnp.float32)),
    mesh=plsc.VectorSubcoreMesh(**_MESH),
    compiler_params=pltpu.CompilerParams(use_tc_tiling_on_sc=False),
    scratch_types=[
        pltpu.VMEM((GK,), jnp.int32),
        pltpu.VMEM((GK,), jnp.int32),
        pltpu.VMEM((GK, H), jnp.float32),
        pltpu.VMEM((GK, H), jnp.float32),
        pltpu.SemaphoreType.DMA,
        pltpu.SemaphoreType.DMA,
        pltpu.SemaphoreType.DMA,
        pltpu.SemaphoreType.DMA,
        pltpu.SemaphoreType.DMA,
    ],
)

BM = 1000


def _ln_tail(h, g_ref, b_ref):
  mu = jnp.mean(h, axis=1, keepdims=True)
  var = jnp.mean((h - mu) ** 2, axis=1, keepdims=True)
  return (h - mu) / jnp.sqrt(var + 1e-5) * g_ref[...] + b_ref[...]


def _b16(a):
  return a.astype(jnp.bfloat16)


def _mm(a, b):
  return jnp.dot(_b16(a), _b16(b), preferred_element_type=jnp.float32)


def _dense0_body(x_ref, agg_ref, w1_ref, b1_ref, w2_ref, b2_ref, eps_ref,
                 g_ref, b_ref, out_ref):
  eps = eps_ref[0, 0]
  u = (1.0 + eps) * x_ref[...] + agg_ref[...]
  ub = _b16(u).astype(jnp.float32)
  w1b = _b16(w1_ref[...]).astype(jnp.float32)
  pre = jnp.maximum(
      ub[:, 0:1] * w1b[0:1, :] + ub[:, 1:2] * w1b[1:2, :] + b1_ref[...],
      0.0)
  h = _mm(pre, w2_ref[...]) + b2_ref[...]
  out_ref[...] = _ln_tail(h, g_ref, b_ref)


def _dense1_body(x_ref, agg_ref, w1_ref, b1_ref, w2_ref, b2_ref, eps_ref,
                 g_ref, b_ref, out_ref):
  eps = eps_ref[0, 0]
  u = (1.0 + eps) * x_ref[...] + agg_ref[...]
  pre = jnp.maximum(_mm(u, w1_ref[...]) + b1_ref[...], 0.0)
  h = _mm(pre, w2_ref[...]) + b2_ref[...]
  out_ref[...] = _ln_tail(h, g_ref, b_ref)


def _make_dense(body, din, w1shape):
  full = lambda shp: pl.BlockSpec(shp, lambda i: (0, 0))
  return pl.pallas_call(
      body,
      grid=(N // BM,),
      in_specs=[
          pl.BlockSpec((BM, din), lambda i: (i, 0)),
          pl.BlockSpec((BM, din), lambda i: (i, 0)),
          full(w1shape),
          full((1, H)),
          full((H, H)),
          full((1, H)),
          full((1, 1)),
          full((1, H)),
          full((1, H)),
      ],
      out_specs=pl.BlockSpec((BM, H), lambda i: (i, 0)),
      out_shape=jax.ShapeDtypeStruct((N, H), jnp.float32),
  )


_dense16 = _make_dense(_dense0_body, 2, (8, H))
_dense64 = _make_dense(_dense1_body, H, (H, H))


def _head_body(hs_ref, hd_ref, wa_ref, wb_ref, b1_ref, w2_ref, b2_ref,
               w3_ref, b3_ref, out_ref):
  z = jnp.maximum(
      _mm(hs_ref[...], wa_ref[...]) + _mm(hd_ref[...], wb_ref[...])
      + b1_ref[...], 0.0)
  z = jnp.maximum(_mm(z, w2_ref[...]) + b2_ref[...], 0.0)
  zb = _b16(z).astype(jnp.float32)
  w3row = _b16(w3_ref[0:1, :]).astype(jnp.float32)
  out_ref[...] = jnp.sum(zb * w3row, axis=1, keepdims=True) + b3_ref[...]


_head = pl.pallas_call(
    _head_body,
    grid=(EL // BM,),
    in_specs=[
        pl.BlockSpec((BM, H), lambda i: (i, 0)),
        pl.BlockSpec((BM, H), lambda i: (i, 0)),
        pl.BlockSpec((H, H), lambda i: (0, 0)),
        pl.BlockSpec((H, H), lambda i: (0, 0)),
        pl.BlockSpec((1, H), lambda i: (0, 0)),
        pl.BlockSpec((H, H // 2), lambda i: (0, 0)),
        pl.BlockSpec((1, H // 2), lambda i: (0, 0)),
        pl.BlockSpec((8, H // 2), lambda i: (0, 0)),
        pl.BlockSpec((1, 1), lambda i: (0, 0)),
    ],
    out_specs=pl.BlockSpec((BM, 1), lambda i: (i, 0)),
    out_shape=jax.ShapeDtypeStruct((EL, 1), jnp.float32),
)


def kernel(x, edge_index, edge_label_index, w1, b1, w2, b2, eps0, ln0_g,
           ln0_b, w3, b3, w4, b4, eps1, ln1_g, ln1_b, we1, be1, we2, be2,
           we3, be3):
  src = jnp.concatenate([edge_index[0],
                         jnp.zeros((E_PAD - E,), jnp.int32)])
  dst = jnp.concatenate([edge_index[1],
                         jnp.full((E_PAD - E,), N, jnp.int32)])
  w1p = jnp.pad(w1, ((0, 6), (0, 0)))

  agg0_raw = _agg0(x.T, src, dst)
  agg0 = agg0_raw.reshape(2, A0_R * 16)[:, :N].T
  h0 = _dense16(x, agg0, w1p, b1.reshape(1, H), w2, b2.reshape(1, H),
                eps0.reshape(1, 1), ln0_g.reshape(1, H), ln0_b.reshape(1, H))
  agg1 = _segsum64(h0, src, dst)
  h1 = _dense64(h0, agg1, w3, b3.reshape(1, H), w4, b4.reshape(1, H),
                eps1.reshape(1, 1), ln1_g.reshape(1, H), ln1_b.reshape(1, H))
  hs, hd = _pair_gather(h1, edge_label_index[0], edge_label_index[1])
  w3p = jnp.pad(we3.T, ((0, 7), (0, 0)))
  logits = _head(hs, hd, we1[:H], we1[H:], be1.reshape(1, H), we2,
                 be2.reshape(1, H // 2), w3p, be3.reshape(1, 1))
  return logits[:, 0]


# speedup vs baseline: 4.3145x; 1.0506x over previous
import functools

import jax
import jax.numpy as jnp
from jax import lax
from jax.experimental import pallas as pl
from jax.experimental.pallas import tpu as pltpu
from jax.experimental.pallas import tpu_sc as plsc

N = 50000
E = 800000
EL = 100000
H = 64

HALF = N // 2
HALF_P = 25040
ZC = 40
N_Z = HALF_P // ZC
K = 128
CPB = 8
IB = K * CPB
NBLK = 50
NPAIR = NBLK // 2
NCH = NBLK * CPB
EPT_E = NCH * K
E_PAD = 16 * EPT_E
OC = 40
N_OUT = HALF // OC

_MESH = dict(core_axis_name="c", subcore_axis_name="s", num_cores=2,
             num_subcores=16)


def _make_segsum(D):
  def body(tab, src, dst, out, agg_sh, sblk0, sblk1, dblk0, dblk1,
           ldst0, ldst1, rows0, rows1, zbuf, obuf, gsem, ssem, isem0, isem1):
    c = lax.axis_index("c")
    s = lax.axis_index("s")
    base = c * HALF
    ebase = s * EPT_E
    sblk = (sblk0, sblk1)
    dblk = (dblk0, dblk1)
    ldst = (ldst0, ldst1)
    rows = (rows0, rows1)
    isem = (isem0, isem1)

    def zrow(r, carry):
      for j in range(D // 16):
        zbuf[r, pl.ds(j * 16, 16)] = jnp.zeros((16,), jnp.float32)
      return carry

    lax.fori_loop(0, ZC, zrow, 0)
    for m in range(40):
      cid = m * 16 + s

      @pl.when(cid < N_Z)
      def _():
        off = pl.multiple_of(cid * ZC, 8)
        pltpu.sync_copy(zbuf, agg_sh.at[pl.ds(off, ZC), :])
    plsc.subcore_barrier()

    def issue_idx(bb, blk):
      off = pl.multiple_of(ebase + blk * IB, 8)
      pltpu.async_copy(src.at[pl.ds(off, IB)], sblk[bb], isem[bb])
      pltpu.async_copy(dst.at[pl.ds(off, IB)], dblk[bb], isem[bb])

    def wait_idx(bb):
      pltpu.make_async_copy(src.at[pl.ds(0, IB)], sblk[bb], isem[bb]).wait()
      pltpu.make_async_copy(dst.at[pl.ds(0, IB)], dblk[bb], isem[bb]).wait()

    def transform(bb, kk, rb):
      for j in range(K // 16):
        d = dblk[bb][pl.ds(kk * K + j * 16, 16)]
        l = d - base
        valid = (l >= 0) & (l < HALF)
        dummy = HALF + lax.bitwise_and(d, 31)
        ldst[rb][pl.ds(j * 16, 16)] = jnp.where(valid, l, dummy)

    def issue_gather(bb, kk, rb):
      pltpu.async_copy(tab.at[sblk[bb].at[pl.ds(kk * K, K)]], rows[rb], gsem)

    def wait_gather(rb):
      pltpu.make_async_copy(tab.at[sblk[0].at[pl.ds(0, K)]], rows[rb],
                            gsem).wait()

    def issue_scatter(rb):
      pltpu.async_copy(rows[rb], agg_sh.at[ldst[rb]], ssem, add=True)

    def wait_scatter(rb):
      pltpu.make_async_copy(rows[rb], agg_sh.at[ldst[rb]], ssem).wait()

    issue_idx(0, 0)
    issue_idx(1, 1)
    wait_idx(0)
    transform(0, 0, 0)
    issue_gather(0, 0, 0)

    def pair_body(i, carry):
      for k in range(16):
        bb = 0 if k < 8 else 1
        kk = k % 8
        rb = k % 2
        nrb = 1 - rb
        if k == 0:
          @pl.when(i >= 1)
          def _():
            wait_scatter(nrb)
        else:
          wait_scatter(nrb)
        wait_gather(rb)
        issue_scatter(rb)
        if kk == 7:
          nblk = 2 * i + 2 + bb

          @pl.when(nblk < NBLK)
          def _():
            issue_idx(bb, nblk)
        if k == 15:
          @pl.when(i < NPAIR - 1)
          def _():
            wait_idx(0)
            transform(0, 0, nrb)
            issue_gather(0, 0, nrb)
        elif k == 7:
          wait_idx(1)
          transform(1, 0, nrb)
          issue_gather(1, 0, nrb)
        else:
          b2 = 0 if k < 7 else 1
          transform(b2, kk + 1, nrb)
          issue_gather(b2, kk + 1, nrb)
      return carry

    lax.fori_loop(0, NPAIR, pair_body, 0)
    wait_scatter(1)
    plsc.subcore_barrier()

    for m in range(40):
      cid = m * 16 + s

      @pl.when(cid < N_OUT)
      def _():
        off = pl.multiple_of(cid * OC, 8)
        pltpu.sync_copy(agg_sh.at[pl.ds(off, OC), :], obuf)
        goff = pl.multiple_of(base + cid * OC, 8)
        pltpu.sync_copy(obuf, out.at[pl.ds(goff, OC), :])

  return pl.kernel(
      body,
      out_type=jax.ShapeDtypeStruct((N, D), jnp.float32),
      mesh=plsc.VectorSubcoreMesh(**_MESH),
      compiler_params=pltpu.CompilerParams(use_tc_tiling_on_sc=False),
      scratch_types=[
          pltpu.VMEM_SHARED((HALF_P, D), jnp.float32),
          pltpu.VMEM((IB,), jnp.int32),
          pltpu.VMEM((IB,), jnp.int32),
          pltpu.VMEM((IB,), jnp.int32),
          pltpu.VMEM((IB,), jnp.int32),
          pltpu.VMEM((K,), jnp.int32),
          pltpu.VMEM((K,), jnp.int32),
          pltpu.VMEM((K, D), jnp.float32),
          pltpu.VMEM((K, D), jnp.float32),
          pltpu.VMEM((ZC, D), jnp.float32),
          pltpu.VMEM((OC, D), jnp.float32),
          pltpu.SemaphoreType.DMA,
          pltpu.SemaphoreType.DMA,
          pltpu.SemaphoreType.DMA,
          pltpu.SemaphoreType.DMA,
      ],
  )


_segsum64 = _make_segsum(H)

A0_R = 3200
A0_RC = A0_R // 128
A0_BLK = 1024
A0_NB = 50
EPT0 = A0_NB * A0_BLK


def _agg0_body(xT, src, dst, out, agg_sh, xcol, acc, sblk0, sblk1, dblk0,
               dblk1, idxall, zbuf, isem0, isem1, ssem):
  c = lax.axis_index("c")
  s = lax.axis_index("s")
  sblk = (sblk0, sblk1)
  dblk = (dblk0, dblk1)
  isem = (isem0, isem1)
  ebase = s * EPT0

  pltpu.async_copy(src.at[pl.ds(pl.multiple_of(ebase, 8), A0_BLK)], sblk0,
                   isem0)
  pltpu.async_copy(dst.at[pl.ds(pl.multiple_of(ebase, 8), A0_BLK)], dblk0,
                   isem0)
  pltpu.sync_copy(xT.at[c], xcol)

  def zacc(r, carry):
    acc[r, pl.ds(0, 16)] = jnp.zeros((16,), jnp.float32)
    return carry

  lax.fori_loop(0, A0_R, zacc, 0)

  def zrow(r, carry):
    zbuf[r, pl.ds(0, 16)] = jnp.zeros((16,), jnp.float32)
    return carry

  lax.fori_loop(0, A0_R // 16, zrow, 0)
  pltpu.sync_copy(zbuf, agg_sh.at[pl.ds(s * (A0_R // 16), A0_R // 16), :])

  lane = lax.iota(jnp.int32, 16)
  for r in range(A0_RC):
    for j in range(8):
      idxall[r, pl.ds(j * 16, 16)] = lane + (r * 128 + j * 16)

  def issue_blk(p, b):
    off = pl.multiple_of(ebase + b * A0_BLK, 8)
    pltpu.async_copy(src.at[pl.ds(off, A0_BLK)], sblk[p], isem[p])
    pltpu.async_copy(dst.at[pl.ds(off, A0_BLK)], dblk[p], isem[p])

  def wait_blk(p):
    pltpu.make_async_copy(src.at[pl.ds(0, A0_BLK)], sblk[p], isem[p]).wait()
    pltpu.make_async_copy(dst.at[pl.ds(0, A0_BLK)], dblk[p], isem[p]).wait()

  def process(p):
    def step(j, carry):
      s16 = sblk[p][pl.ds(j * 16, 16)]
      d16 = dblk[p][pl.ds(j * 16, 16)]
      v = plsc.load_gather(xcol, [s16])
      row = lax.shift_right_logical(d16, 4)
      col = lax.bitwise_and(d16, 15)
      plsc.addupdate_scatter(acc, [row, col], v)
      return carry

    lax.fori_loop(0, A0_BLK // 16, step, 0)

  def blkpair(i, carry):
    for p in (0, 1):
      wait_blk(p)
      if p == 0:
        issue_blk(1, 2 * i + 1)
      else:
        @pl.when(i < A0_NB // 2 - 1)
        def _():
          issue_blk(0, 2 * i + 2)
      process(p)
    return carry

  lax.fori_loop(0, A0_NB // 2, blkpair, 0)

  plsc.subcore_barrier()
  for r in range(A0_RC):
    pltpu.async_copy(acc.at[pl.ds(r * 128, 128), :],
                     agg_sh.at[idxall.at[r]], ssem, add=True)
  for r in range(A0_RC):
    pltpu.make_async_copy(acc.at[pl.ds(0, 128), :],
                          agg_sh.at[idxall.at[0]], ssem).wait()
  plsc.subcore_barrier()
  pltpu.sync_copy(agg_sh.at[pl.ds(s * (A0_R // 16), A0_R // 16), :], zbuf)
  pltpu.sync_copy(zbuf, out.at[c, pl.ds(s * (A0_R // 16), A0_R // 16), :])


_agg0 = pl.kernel(
    _agg0_body,
    out_type=jax.ShapeDtypeStruct((2, A0_R, 16), jnp.float32),
    mesh=plsc.VectorSubcoreMesh(**_MESH),
    compiler_params=pltpu.CompilerParams(use_tc_tiling_on_sc=False,
                                         needs_layout_passes=False),
    scratch_types=[
        pltpu.VMEM_SHARED((A0_R, 16), jnp.float32),
        pltpu.VMEM((N,), jnp.float32),
        pltpu.VMEM((A0_R, 16), jnp.float32),
        pltpu.VMEM((A0_BLK,), jnp.int32),
        pltpu.VMEM((A0_BLK,), jnp.int32),
        pltpu.VMEM((A0_BLK,), jnp.int32),
        pltpu.VMEM((A0_BLK,), jnp.int32),
        pltpu.VMEM((A0_RC, 128), jnp.int32),
        pltpu.VMEM((A0_R // 16, 16), jnp.float32),
        pltpu.SemaphoreType.DMA,
        pltpu.SemaphoreType.DMA,
        pltpu.SemaphoreType.DMA,
    ],
)

GK = 80
_GCH = EL // GK
_GSLOT = (_GCH + 31) // 32


def _pair_gather_body(tab, sidx, didx, hs, hd, ibuf0, ibuf1, rows0, rows1,
                      gsem, wsem0, wsem1, isem0, isem1):
  c = lax.axis_index("c")
  s = lax.axis_index("s")
  wid = s * 2 + c
  ibuf = (ibuf0, ibuf1)
  rows = (rows0, rows1)
  wsem = (wsem0, wsem1)
  isem = (isem0, isem1)
  idx_in = (sidx, didx)
  out = (hs, hd)

  def issue_idx(q, cid):
    off = pl.multiple_of(cid * GK, 8)
    pltpu.async_copy(idx_in[q].at[pl.ds(off, GK)], ibuf[q], isem[q])

  issue_idx(0, wid)
  issue_idx(1, wid)

  def slot(m, carry):
    cid = m * 32 + wid

    for q in (0, 1):
      @pl.when(cid < _GCH)
      def _():
        pltpu.make_async_copy(idx_in[q].at[pl.ds(0, GK)], ibuf[q],
                              isem[q]).wait()

        @pl.when(m >= 1)
        def _():
          pltpu.make_async_copy(rows[q], out[q].at[pl.ds(0, GK), :],
                                wsem[q]).wait()
        pltpu.async_copy(tab.at[ibuf[q]], rows[q], gsem).wait()
        off = pl.multiple_of(cid * GK, 8)
        pltpu.async_copy(rows[q], out[q].at[pl.ds(off, GK), :], wsem[q])

      @pl.when(cid + 32 < _GCH)
      def _():
        issue_idx(q, cid + 32)
    return carry

  lax.fori_loop(0, _GSLOT, slot, 0)
  pltpu.make_async_copy(rows0, hs.at[pl.ds(0, GK), :], wsem0).wait()
  pltpu.make_async_copy(rows1, hd.at[pl.ds(0, GK), :], wsem1).wait()


_pair_gather = pl.kernel(
    _pair_gather_body,
    out_type=(jax.ShapeDtypeStruct((EL, H), jnp.float32),
              jax.ShapeDtypeStruct((EL, H), jnp.float32)),
    mesh=plsc.VectorSubcoreMesh(**_MESH),
    compiler_params=pltpu.CompilerParams(use_tc_tiling_on_sc=False),
    scratch_types=[
        pltpu.VMEM((GK,), jnp.int32),
        pltpu.VMEM((GK,), jnp.int32),
        pltpu.VMEM((GK, H), jnp.float32),
        pltpu.VMEM((GK, H), jnp.float32),
        pltpu.SemaphoreType.DMA,
        pltpu.SemaphoreType.DMA,
        pltpu.SemaphoreType.DMA,
        pltpu.SemaphoreType.DMA,
        pltpu.SemaphoreType.DMA,
    ],
)

BM = 1000


def _ln_tail(h, g_ref, b_ref):
  mu = jnp.mean(h, axis=1, keepdims=True)
  var = jnp.mean((h - mu) ** 2, axis=1, keepdims=True)
  return (h - mu) / jnp.sqrt(var + 1e-5) * g_ref[...] + b_ref[...]


def _b16(a):
  return a.astype(jnp.bfloat16)


def _mm(a, b):
  return jnp.dot(_b16(a), _b16(b), preferred_element_type=jnp.float32)


def _dense0_body(x_ref, agg_ref, w1_ref, b1_ref, w2_ref, b2_ref, eps_ref,
                 g_ref, b_ref, out_ref):
  eps = eps_ref[0, 0]
  u = (1.0 + eps) * x_ref[...] + agg_ref[...]
  ub = _b16(u).astype(jnp.float32)
  w1b = _b16(w1_ref[...]).astype(jnp.float32)
  pre = jnp.maximum(
      ub[:, 0:1] * w1b[0:1, :] + ub[:, 1:2] * w1b[1:2, :] + b1_ref[...],
      0.0)
  h = _mm(pre, w2_ref[...]) + b2_ref[...]
  out_ref[...] = _ln_tail(h, g_ref, b_ref)


def _dense1_body(x_ref, agg_ref, w1_ref, b1_ref, w2_ref, b2_ref, eps_ref,
                 g_ref, b_ref, out_ref):
  eps = eps_ref[0, 0]
  u = (1.0 + eps) * x_ref[...] + agg_ref[...]
  pre = jnp.maximum(_mm(u, w1_ref[...]) + b1_ref[...], 0.0)
  h = _mm(pre, w2_ref[...]) + b2_ref[...]
  out_ref[...] = _ln_tail(h, g_ref, b_ref)


def _make_dense(body, din, w1shape):
  full = lambda shp: pl.BlockSpec(shp, lambda i: (0, 0))
  return pl.pallas_call(
      body,
      grid=(N // BM,),
      in_specs=[
          pl.BlockSpec((BM, din), lambda i: (i, 0)),
          pl.BlockSpec((BM, din), lambda i: (i, 0)),
          full(w1shape),
          full((1, H)),
          full((H, H)),
          full((1, H)),
          full((1, 1)),
          full((1, H)),
          full((1, H)),
      ],
      out_specs=pl.BlockSpec((BM, H), lambda i: (i, 0)),
      out_shape=jax.ShapeDtypeStruct((N, H), jnp.float32),
  )


_dense16 = _make_dense(_dense0_body, 2, (8, H))
_dense64 = _make_dense(_dense1_body, H, (H, H))


def _head_body(hs_ref, hd_ref, wa_ref, wb_ref, b1_ref, w2_ref, b2_ref,
               w3_ref, b3_ref, out_ref):
  z = jnp.maximum(
      _mm(hs_ref[...], wa_ref[...]) + _mm(hd_ref[...], wb_ref[...])
      + b1_ref[...], 0.0)
  z = jnp.maximum(_mm(z, w2_ref[...]) + b2_ref[...], 0.0)
  zb = _b16(z).astype(jnp.float32)
  w3row = _b16(w3_ref[0:1, :]).astype(jnp.float32)
  out_ref[...] = jnp.sum(zb * w3row, axis=1, keepdims=True) + b3_ref[...]


_head = pl.pallas_call(
    _head_body,
    grid=(EL // BM,),
    in_specs=[
        pl.BlockSpec((BM, H), lambda i: (i, 0)),
        pl.BlockSpec((BM, H), lambda i: (i, 0)),
        pl.BlockSpec((H, H), lambda i: (0, 0)),
        pl.BlockSpec((H, H), lambda i: (0, 0)),
        pl.BlockSpec((1, H), lambda i: (0, 0)),
        pl.BlockSpec((H, H // 2), lambda i: (0, 0)),
        pl.BlockSpec((1, H // 2), lambda i: (0, 0)),
        pl.BlockSpec((8, H // 2), lambda i: (0, 0)),
        pl.BlockSpec((1, 1), lambda i: (0, 0)),
    ],
    out_specs=pl.BlockSpec((BM, 1), lambda i: (i, 0)),
    out_shape=jax.ShapeDtypeStruct((EL, 1), jnp.float32),
)


def kernel(x, edge_index, edge_label_index, w1, b1, w2, b2, eps0, ln0_g,
           ln0_b, w3, b3, w4, b4, eps1, ln1_g, ln1_b, we1, be1, we2, be2,
           we3, be3):
  src = jnp.concatenate([edge_index[0],
                         jnp.zeros((E_PAD - E,), jnp.int32)])
  dst = jnp.concatenate([edge_index[1],
                         jnp.full((E_PAD - E,), N, jnp.int32)])
  w1p = jnp.pad(w1, ((0, 6), (0, 0)))

  agg0_raw = _agg0(x.T, src, dst)
  agg0 = agg0_raw.reshape(2, A0_R * 16)[:, :N].T
  h0 = _dense16(x, agg0, w1p, b1.reshape(1, H), w2, b2.reshape(1, H),
                eps0.reshape(1, 1), ln0_g.reshape(1, H), ln0_b.reshape(1, H))
  agg1 = _segsum64(h0, src, dst)
  h1 = _dense64(h0, agg1, w3, b3.reshape(1, H), w4, b4.reshape(1, H),
                eps1.reshape(1, 1), ln1_g.reshape(1, H), ln1_b.reshape(1, H))
  hs, hd = _pair_gather(h1, edge_label_index[0], edge_label_index[1])
  w3p = jnp.pad(we3.T, ((0, 7), (0, 0)))
  logits = _head(hs, hd, we1[:H], we1[H:], be1.reshape(1, H), we2,
                 be2.reshape(1, H // 2), w3p, be3.reshape(1, 1))
  return logits[:, 0]


# speedup vs baseline: 4.4905x; 1.0408x over previous
import functools

import jax
import jax.numpy as jnp
from jax import lax
from jax.experimental import pallas as pl
from jax.experimental.pallas import tpu as pltpu
from jax.experimental.pallas import tpu_sc as plsc

N = 50000
E = 800000
EL = 100000
H = 64

HALF = N // 2
HALF_P = 25040
ZC = 40
N_Z = HALF_P // ZC
K = 128
CPB = 8
IB = K * CPB
NBLK = 50
NPAIR = NBLK // 2
NCH = NBLK * CPB
EPT_E = NCH * K
E_PAD = 16 * EPT_E
OC = 40
N_OUT = HALF // OC

_MESH = dict(core_axis_name="c", subcore_axis_name="s", num_cores=2,
             num_subcores=16)


def _make_segsum(D):
  def body(tab, src, dst, out, agg_sh, sblk0, sblk1, dblk0, dblk1,
           ldst0, ldst1, rows0, rows1, zbuf, obuf, gsem, ssem, isem0, isem1):
    c = lax.axis_index("c")
    s = lax.axis_index("s")
    base = c * HALF
    ebase = s * EPT_E
    sblk = (sblk0, sblk1)
    dblk = (dblk0, dblk1)
    ldst = (ldst0, ldst1)
    rows = (rows0, rows1)
    isem = (isem0, isem1)

    def zrow(r, carry):
      for j in range(D // 16):
        zbuf[r, pl.ds(j * 16, 16)] = jnp.zeros((16,), jnp.float32)
      return carry

    lax.fori_loop(0, ZC, zrow, 0)
    for m in range(40):
      cid = m * 16 + s

      @pl.when(cid < N_Z)
      def _():
        off = pl.multiple_of(cid * ZC, 8)
        pltpu.sync_copy(zbuf, agg_sh.at[pl.ds(off, ZC), :])
    plsc.subcore_barrier()

    def issue_idx(bb, blk):
      off = pl.multiple_of(ebase + blk * IB, 8)
      pltpu.async_copy(src.at[pl.ds(off, IB)], sblk[bb], isem[bb])
      pltpu.async_copy(dst.at[pl.ds(off, IB)], dblk[bb], isem[bb])

    def wait_idx(bb):
      pltpu.make_async_copy(src.at[pl.ds(0, IB)], sblk[bb], isem[bb]).wait()
      pltpu.make_async_copy(dst.at[pl.ds(0, IB)], dblk[bb], isem[bb]).wait()

    def transform(bb, kk, rb):
      for j in range(K // 16):
        d = dblk[bb][pl.ds(kk * K + j * 16, 16)]
        l = d - base
        valid = (l >= 0) & (l < HALF)
        dummy = HALF + lax.bitwise_and(d, 31)
        ldst[rb][pl.ds(j * 16, 16)] = jnp.where(valid, l, dummy)

    def issue_gather(bb, kk, rb):
      pltpu.async_copy(tab.at[sblk[bb].at[pl.ds(kk * K, K)]], rows[rb], gsem)

    def wait_gather(rb):
      pltpu.make_async_copy(tab.at[sblk[0].at[pl.ds(0, K)]], rows[rb],
                            gsem).wait()

    def issue_scatter(rb):
      pltpu.async_copy(rows[rb], agg_sh.at[ldst[rb]], ssem, add=True)

    def wait_scatter(rb):
      pltpu.make_async_copy(rows[rb], agg_sh.at[ldst[rb]], ssem).wait()

    issue_idx(0, 0)
    issue_idx(1, 1)
    wait_idx(0)
    transform(0, 0, 0)
    issue_gather(0, 0, 0)

    def pair_body(i, carry):
      for k in range(16):
        bb = 0 if k < 8 else 1
        kk = k % 8
        rb = k % 2
        nrb = 1 - rb
        if k == 0:
          @pl.when(i >= 1)
          def _():
            wait_scatter(nrb)
        else:
          wait_scatter(nrb)
        if k == 15:
          @pl.when(i < NPAIR - 1)
          def _():
            wait_idx(0)
            transform(0, 0, nrb)
            issue_gather(0, 0, nrb)
        elif k == 7:
          wait_idx(1)
          transform(1, 0, nrb)
          issue_gather(1, 0, nrb)
        else:
          b2 = 0 if k < 7 else 1
          transform(b2, kk + 1, nrb)
          issue_gather(b2, kk + 1, nrb)
        wait_gather(rb)
        issue_scatter(rb)
        if kk == 7:
          nblk = 2 * i + 2 + bb

          @pl.when(nblk < NBLK)
          def _():
            issue_idx(bb, nblk)
      return carry

    lax.fori_loop(0, NPAIR, pair_body, 0)
    wait_scatter(1)
    plsc.subcore_barrier()

    for m in range(40):
      cid = m * 16 + s

      @pl.when(cid < N_OUT)
      def _():
        off = pl.multiple_of(cid * OC, 8)
        pltpu.sync_copy(agg_sh.at[pl.ds(off, OC), :], obuf)
        goff = pl.multiple_of(base + cid * OC, 8)
        pltpu.sync_copy(obuf, out.at[pl.ds(goff, OC), :])

  return pl.kernel(
      body,
      out_type=jax.ShapeDtypeStruct((N, D), jnp.float32),
      mesh=plsc.VectorSubcoreMesh(**_MESH),
      compiler_params=pltpu.CompilerParams(use_tc_tiling_on_sc=False),
      scratch_types=[
          pltpu.VMEM_SHARED((HALF_P, D), jnp.float32),
          pltpu.VMEM((IB,), jnp.int32),
          pltpu.VMEM((IB,), jnp.int32),
          pltpu.VMEM((IB,), jnp.int32),
          pltpu.VMEM((IB,), jnp.int32),
          pltpu.VMEM((K,), jnp.int32),
          pltpu.VMEM((K,), jnp.int32),
          pltpu.VMEM((K, D), jnp.float32),
          pltpu.VMEM((K, D), jnp.float32),
          pltpu.VMEM((ZC, D), jnp.float32),
          pltpu.VMEM((OC, D), jnp.float32),
          pltpu.SemaphoreType.DMA,
          pltpu.SemaphoreType.DMA,
          pltpu.SemaphoreType.DMA,
          pltpu.SemaphoreType.DMA,
      ],
  )


_segsum64 = _make_segsum(H)

A0_R = 3200
A0_RC = A0_R // 128
A0_BLK = 1024
A0_NB = 50
EPT0 = A0_NB * A0_BLK


def _agg0_body(xT, src, dst, out, agg_sh, xcol, acc, sblk0, sblk1, dblk0,
               dblk1, idxall, zbuf, isem0, isem1, ssem):
  c = lax.axis_index("c")
  s = lax.axis_index("s")
  sblk = (sblk0, sblk1)
  dblk = (dblk0, dblk1)
  isem = (isem0, isem1)
  ebase = s * EPT0

  pltpu.async_copy(src.at[pl.ds(pl.multiple_of(ebase, 8), A0_BLK)], sblk0,
                   isem0)
  pltpu.async_copy(dst.at[pl.ds(pl.multiple_of(ebase, 8), A0_BLK)], dblk0,
                   isem0)
  pltpu.sync_copy(xT.at[c], xcol)

  def zacc(r, carry):
    acc[r, pl.ds(0, 16)] = jnp.zeros((16,), jnp.float32)
    return carry

  lax.fori_loop(0, A0_R, zacc, 0)

  def zrow(r, carry):
    zbuf[r, pl.ds(0, 16)] = jnp.zeros((16,), jnp.float32)
    return carry

  lax.fori_loop(0, A0_R // 16, zrow, 0)
  pltpu.sync_copy(zbuf, agg_sh.at[pl.ds(s * (A0_R // 16), A0_R // 16), :])

  lane = lax.iota(jnp.int32, 16)
  for r in range(A0_RC):
    for j in range(8):
      idxall[r, pl.ds(j * 16, 16)] = lane + (r * 128 + j * 16)

  def issue_blk(p, b):
    off = pl.multiple_of(ebase + b * A0_BLK, 8)
    pltpu.async_copy(src.at[pl.ds(off, A0_BLK)], sblk[p], isem[p])
    pltpu.async_copy(dst.at[pl.ds(off, A0_BLK)], dblk[p], isem[p])

  def wait_blk(p):
    pltpu.make_async_copy(src.at[pl.ds(0, A0_BLK)], sblk[p], isem[p]).wait()
    pltpu.make_async_copy(dst.at[pl.ds(0, A0_BLK)], dblk[p], isem[p]).wait()

  def process(p):
    def step(j, carry):
      s16 = sblk[p][pl.ds(j * 16, 16)]
      d16 = dblk[p][pl.ds(j * 16, 16)]
      v = plsc.load_gather(xcol, [s16])
      row = lax.shift_right_logical(d16, 4)
      col = lax.bitwise_and(d16, 15)
      plsc.addupdate_scatter(acc, [row, col], v)
      return carry

    lax.fori_loop(0, A0_BLK // 16, step, 0)

  def blkpair(i, carry):
    for p in (0, 1):
      wait_blk(p)
      if p == 0:
        issue_blk(1, 2 * i + 1)
      else:
        @pl.when(i < A0_NB // 2 - 1)
        def _():
          issue_blk(0, 2 * i + 2)
      process(p)
    return carry

  lax.fori_loop(0, A0_NB // 2, blkpair, 0)

  plsc.subcore_barrier()
  for r in range(A0_RC):
    pltpu.async_copy(acc.at[pl.ds(r * 128, 128), :],
                     agg_sh.at[idxall.at[r]], ssem, add=True)
  for r in range(A0_RC):
    pltpu.make_async_copy(acc.at[pl.ds(0, 128), :],
                          agg_sh.at[idxall.at[0]], ssem).wait()
  plsc.subcore_barrier()
  pltpu.sync_copy(agg_sh.at[pl.ds(s * (A0_R // 16), A0_R // 16), :], zbuf)
  pltpu.sync_copy(zbuf, out.at[c, pl.ds(s * (A0_R // 16), A0_R // 16), :])


_agg0 = pl.kernel(
    _agg0_body,
    out_type=jax.ShapeDtypeStruct((2, A0_R, 16), jnp.float32),
    mesh=plsc.VectorSubcoreMesh(**_MESH),
    compiler_params=pltpu.CompilerParams(use_tc_tiling_on_sc=False,
                                         needs_layout_passes=False),
    scratch_types=[
        pltpu.VMEM_SHARED((A0_R, 16), jnp.float32),
        pltpu.VMEM((N,), jnp.float32),
        pltpu.VMEM((A0_R, 16), jnp.float32),
        pltpu.VMEM((A0_BLK,), jnp.int32),
        pltpu.VMEM((A0_BLK,), jnp.int32),
        pltpu.VMEM((A0_BLK,), jnp.int32),
        pltpu.VMEM((A0_BLK,), jnp.int32),
        pltpu.VMEM((A0_RC, 128), jnp.int32),
        pltpu.VMEM((A0_R // 16, 16), jnp.float32),
        pltpu.SemaphoreType.DMA,
        pltpu.SemaphoreType.DMA,
        pltpu.SemaphoreType.DMA,
    ],
)

GK = 80
_GCH = EL // GK
_GSLOT = (_GCH + 31) // 32


def _pair_gather_body(tab, sidx, didx, hs, hd, ibuf0, ibuf1, rows0, rows1,
                      gsem, wsem0, wsem1, isem0, isem1):
  c = lax.axis_index("c")
  s = lax.axis_index("s")
  wid = s * 2 + c
  ibuf = (ibuf0, ibuf1)
  rows = (rows0, rows1)
  wsem = (wsem0, wsem1)
  isem = (isem0, isem1)
  idx_in = (sidx, didx)
  out = (hs, hd)

  def issue_idx(q, cid):
    off = pl.multiple_of(cid * GK, 8)
    pltpu.async_copy(idx_in[q].at[pl.ds(off, GK)], ibuf[q], isem[q])

  issue_idx(0, wid)
  issue_idx(1, wid)

  def slot(m, carry):
    cid = m * 32 + wid

    for q in (0, 1):
      @pl.when(cid < _GCH)
      def _():
        pltpu.make_async_copy(idx_in[q].at[pl.ds(0, GK)], ibuf[q],
                              isem[q]).wait()

        @pl.when(m >= 1)
        def _():
          pltpu.make_async_copy(rows[q], out[q].at[pl.ds(0, GK), :],
                                wsem[q]).wait()
        pltpu.async_copy(tab.at[ibuf[q]], rows[q], gsem).wait()
        off = pl.multiple_of(cid * GK, 8)
        pltpu.async_copy(rows[q], out[q].at[pl.ds(off, GK), :], wsem[q])

      @pl.when(cid + 32 < _GCH)
      def _():
        issue_idx(q, cid + 32)
    return carry

  lax.fori_loop(0, _GSLOT, slot, 0)
  pltpu.make_async_copy(rows0, hs.at[pl.ds(0, GK), :], wsem0).wait()
  pltpu.make_async_copy(rows1, hd.at[pl.ds(0, GK), :], wsem1).wait()


_pair_gather = pl.kernel(
    _pair_gather_body,
    out_type=(jax.ShapeDtypeStruct((EL, H), jnp.float32),
              jax.ShapeDtypeStruct((EL, H), jnp.float32)),
    mesh=plsc.VectorSubcoreMesh(**_MESH),
    compiler_params=pltpu.CompilerParams(use_tc_tiling_on_sc=False),
    scratch_types=[
        pltpu.VMEM((GK,), jnp.int32),
        pltpu.VMEM((GK,), jnp.int32),
        pltpu.VMEM((GK, H), jnp.float32),
        pltpu.VMEM((GK, H), jnp.float32),
        pltpu.SemaphoreType.DMA,
        pltpu.SemaphoreType.DMA,
        pltpu.SemaphoreType.DMA,
        pltpu.SemaphoreType.DMA,
        pltpu.SemaphoreType.DMA,
    ],
)

BM = 1000


def _ln_tail(h, g_ref, b_ref):
  mu = jnp.mean(h, axis=1, keepdims=True)
  var = jnp.mean((h - mu) ** 2, axis=1, keepdims=True)
  return (h - mu) / jnp.sqrt(var + 1e-5) * g_ref[...] + b_ref[...]


def _b16(a):
  return a.astype(jnp.bfloat16)


def _mm(a, b):
  return jnp.dot(_b16(a), _b16(b), preferred_element_type=jnp.float32)


def _dense0_body(x_ref, agg_ref, w1_ref, b1_ref, w2_ref, b2_ref, eps_ref,
                 g_ref, b_ref, out_ref):
  eps = eps_ref[0, 0]
  u = (1.0 + eps) * x_ref[...] + agg_ref[...]
  ub = _b16(u).astype(jnp.float32)
  w1b = _b16(w1_ref[...]).astype(jnp.float32)
  pre = jnp.maximum(
      ub[:, 0:1] * w1b[0:1, :] + ub[:, 1:2] * w1b[1:2, :] + b1_ref[...],
      0.0)
  h = _mm(pre, w2_ref[...]) + b2_ref[...]
  out_ref[...] = _ln_tail(h, g_ref, b_ref)


def _dense1_body(x_ref, agg_ref, w1_ref, b1_ref, w2_ref, b2_ref, eps_ref,
                 g_ref, b_ref, out_ref):
  eps = eps_ref[0, 0]
  u = (1.0 + eps) * x_ref[...] + agg_ref[...]
  pre = jnp.maximum(_mm(u, w1_ref[...]) + b1_ref[...], 0.0)
  h = _mm(pre, w2_ref[...]) + b2_ref[...]
  out_ref[...] = _ln_tail(h, g_ref, b_ref)


def _make_dense(body, din, w1shape):
  full = lambda shp: pl.BlockSpec(shp, lambda i: (0, 0))
  return pl.pallas_call(
      body,
      grid=(N // BM,),
      in_specs=[
          pl.BlockSpec((BM, din), lambda i: (i, 0)),
          pl.BlockSpec((BM, din), lambda i: (i, 0)),
          full(w1shape),
          full((1, H)),
          full((H, H)),
          full((1, H)),
          full((1, 1)),
          full((1, H)),
          full((1, H)),
      ],
      out_specs=pl.BlockSpec((BM, H), lambda i: (i, 0)),
      out_shape=jax.ShapeDtypeStruct((N, H), jnp.float32),
  )


_dense16 = _make_dense(_dense0_body, 2, (8, H))
_dense64 = _make_dense(_dense1_body, H, (H, H))


def _head_body(hs_ref, hd_ref, wa_ref, wb_ref, b1_ref, w2_ref, b2_ref,
               w3_ref, b3_ref, out_ref):
  z = jnp.maximum(
      _mm(hs_ref[...], wa_ref[...]) + _mm(hd_ref[...], wb_ref[...])
      + b1_ref[...], 0.0)
  z = jnp.maximum(_mm(z, w2_ref[...]) + b2_ref[...], 0.0)
  zb = _b16(z).astype(jnp.float32)
  w3row = _b16(w3_ref[0:1, :]).astype(jnp.float32)
  out_ref[...] = jnp.sum(zb * w3row, axis=1, keepdims=True) + b3_ref[...]


_head = pl.pallas_call(
    _head_body,
    grid=(EL // BM,),
    in_specs=[
        pl.BlockSpec((BM, H), lambda i: (i, 0)),
        pl.BlockSpec((BM, H), lambda i: (i, 0)),
        pl.BlockSpec((H, H), lambda i: (0, 0)),
        pl.BlockSpec((H, H), lambda i: (0, 0)),
        pl.BlockSpec((1, H), lambda i: (0, 0)),
        pl.BlockSpec((H, H // 2), lambda i: (0, 0)),
        pl.BlockSpec((1, H // 2), lambda i: (0, 0)),
        pl.BlockSpec((8, H // 2), lambda i: (0, 0)),
        pl.BlockSpec((1, 1), lambda i: (0, 0)),
    ],
    out_specs=pl.BlockSpec((BM, 1), lambda i: (i, 0)),
    out_shape=jax.ShapeDtypeStruct((EL, 1), jnp.float32),
)


def kernel(x, edge_index, edge_label_index, w1, b1, w2, b2, eps0, ln0_g,
           ln0_b, w3, b3, w4, b4, eps1, ln1_g, ln1_b, we1, be1, we2, be2,
           we3, be3):
  src = jnp.concatenate([edge_index[0],
                         jnp.zeros((E_PAD - E,), jnp.int32)])
  dst = jnp.concatenate([edge_index[1],
                         jnp.full((E_PAD - E,), N, jnp.int32)])
  w1p = jnp.pad(w1, ((0, 6), (0, 0)))

  agg0_raw = _agg0(x.T, src, dst)
  agg0 = agg0_raw.reshape(2, A0_R * 16)[:, :N].T
  h0 = _dense16(x, agg0, w1p, b1.reshape(1, H), w2, b2.reshape(1, H),
                eps0.reshape(1, 1), ln0_g.reshape(1, H), ln0_b.reshape(1, H))
  agg1 = _segsum64(h0, src, dst)
  h1 = _dense64(h0, agg1, w3, b3.reshape(1, H), w4, b4.reshape(1, H),
                eps1.reshape(1, 1), ln1_g.reshape(1, H), ln1_b.reshape(1, H))
  hs, hd = _pair_gather(h1, edge_label_index[0], edge_label_index[1])
  w3p = jnp.pad(we3.T, ((0, 7), (0, 0)))
  logits = _head(hs, hd, we1[:H], we1[H:], be1.reshape(1, H), we2,
                 be2.reshape(1, H // 2), w3p, be3.reshape(1, 1))
  return logits[:, 0]


# speedup vs baseline: 4.4965x; 1.0013x over previous
import jax
import jax.numpy as jnp
from jax import lax
from jax.experimental import pallas as pl
from jax.experimental.pallas import tpu as pltpu
from jax.experimental.pallas import tpu_sc as plsc

N = 50000
E = 800000
EL = 100000
H = 64

HALF = N // 2
HALF_P = 25040
ZC = 40
N_Z = HALF_P // ZC
K = 128
CPB = 8
IB = K * CPB
NBLK = 50
NPAIR = NBLK // 2
NCH = NBLK * CPB
EPT_E = NCH * K
E_PAD = 16 * EPT_E
OC = 40
N_OUT = HALF // OC

_MESH = dict(core_axis_name="c", subcore_axis_name="s", num_cores=2,
             num_subcores=16)


def _make_segsum(D):
  def body(tab, src, dst, out, agg_sh, sblk0, sblk1, dblk0, dblk1,
           ldst0, ldst1, rows0, rows1, zbuf, obuf, gsem, ssem, isem0, isem1):
    c = lax.axis_index("c")
    s = lax.axis_index("s")
    base = c * HALF
    ebase = s * EPT_E
    sblk = (sblk0, sblk1)
    dblk = (dblk0, dblk1)
    ldst = (ldst0, ldst1)
    rows = (rows0, rows1)
    isem = (isem0, isem1)

    def zrow(r, carry):
      for j in range(D // 16):
        zbuf[r, pl.ds(j * 16, 16)] = jnp.zeros((16,), jnp.float32)
      return carry

    lax.fori_loop(0, ZC, zrow, 0)
    for m in range(40):
      cid = m * 16 + s

      @pl.when(cid < N_Z)
      def _():
        off = pl.multiple_of(cid * ZC, 8)
        pltpu.sync_copy(zbuf, agg_sh.at[pl.ds(off, ZC), :])
    plsc.subcore_barrier()

    def issue_idx(bb, blk):
      off = pl.multiple_of(ebase + blk * IB, 8)
      pltpu.async_copy(src.at[pl.ds(off, IB)], sblk[bb], isem[bb])
      pltpu.async_copy(dst.at[pl.ds(off, IB)], dblk[bb], isem[bb])

    def wait_idx(bb):
      pltpu.make_async_copy(src.at[pl.ds(0, IB)], sblk[bb], isem[bb]).wait()
      pltpu.make_async_copy(dst.at[pl.ds(0, IB)], dblk[bb], isem[bb]).wait()

    def transform(bb, kk, rb):
      for j in range(K // 16):
        d = dblk[bb][pl.ds(kk * K + j * 16, 16)]
        l = d - base
        valid = (l >= 0) & (l < HALF)
        dummy = HALF + lax.bitwise_and(d, 31)
        ldst[rb][pl.ds(j * 16, 16)] = jnp.where(valid, l, dummy)

    def issue_gather(bb, kk, rb):
      pltpu.async_copy(tab.at[sblk[bb].at[pl.ds(kk * K, K)]], rows[rb], gsem)

    def wait_gather(rb):
      pltpu.make_async_copy(tab.at[sblk[0].at[pl.ds(0, K)]], rows[rb],
                            gsem).wait()

    def issue_scatter(rb):
      pltpu.async_copy(rows[rb], agg_sh.at[ldst[rb]], ssem, add=True)

    def wait_scatter(rb):
      pltpu.make_async_copy(rows[rb], agg_sh.at[ldst[rb]], ssem).wait()

    issue_idx(0, 0)
    issue_idx(1, 1)
    wait_idx(0)
    transform(0, 0, 0)
    issue_gather(0, 0, 0)

    def pair_body(i, carry):
      for k in range(16):
        bb = 0 if k < 8 else 1
        kk = k % 8
        rb = k % 2
        nrb = 1 - rb
        if k == 0:
          @pl.when(i >= 1)
          def _():
            wait_scatter(nrb)
        else:
          wait_scatter(nrb)
        if k == 15:
          @pl.when(i < NPAIR - 1)
          def _():
            wait_idx(0)
            transform(0, 0, nrb)
            issue_gather(0, 0, nrb)
        elif k == 7:
          wait_idx(1)
          transform(1, 0, nrb)
          issue_gather(1, 0, nrb)
        else:
          b2 = 0 if k < 7 else 1
          transform(b2, kk + 1, nrb)
          issue_gather(b2, kk + 1, nrb)
        wait_gather(rb)
        issue_scatter(rb)
        if kk == 7:
          nblk = 2 * i + 2 + bb

          @pl.when(nblk < NBLK)
          def _():
            issue_idx(bb, nblk)
      return carry

    lax.fori_loop(0, NPAIR, pair_body, 0)
    wait_scatter(1)
    plsc.subcore_barrier()

    for m in range(40):
      cid = m * 16 + s

      @pl.when(cid < N_OUT)
      def _():
        off = pl.multiple_of(cid * OC, 8)
        pltpu.sync_copy(agg_sh.at[pl.ds(off, OC), :], obuf)
        goff = pl.multiple_of(base + cid * OC, 8)
        pltpu.sync_copy(obuf, out.at[pl.ds(goff, OC), :])

  return pl.kernel(
      body,
      out_type=jax.ShapeDtypeStruct((N, D), jnp.float32),
      mesh=plsc.VectorSubcoreMesh(**_MESH),
      compiler_params=pltpu.CompilerParams(use_tc_tiling_on_sc=False),
      scratch_types=[
          pltpu.VMEM_SHARED((HALF_P, D), jnp.float32),
          pltpu.VMEM((IB,), jnp.int32),
          pltpu.VMEM((IB,), jnp.int32),
          pltpu.VMEM((IB,), jnp.int32),
          pltpu.VMEM((IB,), jnp.int32),
          pltpu.VMEM((K,), jnp.int32),
          pltpu.VMEM((K,), jnp.int32),
          pltpu.VMEM((K, D), jnp.float32),
          pltpu.VMEM((K, D), jnp.float32),
          pltpu.VMEM((ZC, D), jnp.float32),
          pltpu.VMEM((OC, D), jnp.float32),
          pltpu.SemaphoreType.DMA,
          pltpu.SemaphoreType.DMA,
          pltpu.SemaphoreType.DMA,
          pltpu.SemaphoreType.DMA,
      ],
  )


_segsum64 = _make_segsum(H)

A0_R = 3200
A0_RC = A0_R // 128
A0_BLK = 1024
A0_NB = 50
EPT0 = A0_NB * A0_BLK


def _agg0_body(xT, src, dst, out, agg_sh, xcol, acc, sblk0, sblk1, dblk0,
               dblk1, idxall, zbuf, isem0, isem1, ssem):
  c = lax.axis_index("c")
  s = lax.axis_index("s")
  sblk = (sblk0, sblk1)
  dblk = (dblk0, dblk1)
  isem = (isem0, isem1)
  ebase = s * EPT0

  pltpu.async_copy(src.at[pl.ds(pl.multiple_of(ebase, 8), A0_BLK)], sblk0,
                   isem0)
  pltpu.async_copy(dst.at[pl.ds(pl.multiple_of(ebase, 8), A0_BLK)], dblk0,
                   isem0)
  pltpu.sync_copy(xT.at[c], xcol)

  def zacc(r, carry):
    acc[r, pl.ds(0, 16)] = jnp.zeros((16,), jnp.float32)
    return carry

  lax.fori_loop(0, A0_R, zacc, 0)

  def zrow(r, carry):
    zbuf[r, pl.ds(0, 16)] = jnp.zeros((16,), jnp.float32)
    return carry

  lax.fori_loop(0, A0_R // 16, zrow, 0)
  pltpu.sync_copy(zbuf, agg_sh.at[pl.ds(s * (A0_R // 16), A0_R // 16), :])

  lane = lax.iota(jnp.int32, 16)
  for r in range(A0_RC):
    for j in range(8):
      idxall[r, pl.ds(j * 16, 16)] = lane + (r * 128 + j * 16)

  def issue_blk(p, b):
    off = pl.multiple_of(ebase + b * A0_BLK, 8)
    pltpu.async_copy(src.at[pl.ds(off, A0_BLK)], sblk[p], isem[p])
    pltpu.async_copy(dst.at[pl.ds(off, A0_BLK)], dblk[p], isem[p])

  def wait_blk(p):
    pltpu.make_async_copy(src.at[pl.ds(0, A0_BLK)], sblk[p], isem[p]).wait()
    pltpu.make_async_copy(dst.at[pl.ds(0, A0_BLK)], dblk[p], isem[p]).wait()

  def process(p):
    def step(j, carry):
      s16 = sblk[p][pl.ds(j * 16, 16)]
      d16 = dblk[p][pl.ds(j * 16, 16)]
      v = plsc.load_gather(xcol, [s16])
      row = lax.shift_right_logical(d16, 4)
      col = lax.bitwise_and(d16, 15)
      plsc.addupdate_scatter(acc, [row, col], v)
      return carry

    lax.fori_loop(0, A0_BLK // 16, step, 0)

  def blkpair(i, carry):
    for p in (0, 1):
      wait_blk(p)
      if p == 0:
        issue_blk(1, 2 * i + 1)
      else:
        @pl.when(i < A0_NB // 2 - 1)
        def _():
          issue_blk(0, 2 * i + 2)
      process(p)
    return carry

  lax.fori_loop(0, A0_NB // 2, blkpair, 0)

  plsc.subcore_barrier()
  for r in range(A0_RC):
    pltpu.async_copy(acc.at[pl.ds(r * 128, 128), :],
                     agg_sh.at[idxall.at[r]], ssem, add=True)
  for r in range(A0_RC):
    pltpu.make_async_copy(acc.at[pl.ds(0, 128), :],
                          agg_sh.at[idxall.at[0]], ssem).wait()
  plsc.subcore_barrier()
  pltpu.sync_copy(agg_sh.at[pl.ds(s * (A0_R // 16), A0_R // 16), :], zbuf)
  pltpu.sync_copy(zbuf, out.at[c, pl.ds(s * (A0_R // 16), A0_R // 16), :])


_agg0 = pl.kernel(
    _agg0_body,
    out_type=jax.ShapeDtypeStruct((2, A0_R, 16), jnp.float32),
    mesh=plsc.VectorSubcoreMesh(**_MESH),
    compiler_params=pltpu.CompilerParams(use_tc_tiling_on_sc=False,
                                         needs_layout_passes=False),
    scratch_types=[
        pltpu.VMEM_SHARED((A0_R, 16), jnp.float32),
        pltpu.VMEM((N,), jnp.float32),
        pltpu.VMEM((A0_R, 16), jnp.float32),
        pltpu.VMEM((A0_BLK,), jnp.int32),
        pltpu.VMEM((A0_BLK,), jnp.int32),
        pltpu.VMEM((A0_BLK,), jnp.int32),
        pltpu.VMEM((A0_BLK,), jnp.int32),
        pltpu.VMEM((A0_RC, 128), jnp.int32),
        pltpu.VMEM((A0_R // 16, 16), jnp.float32),
        pltpu.SemaphoreType.DMA,
        pltpu.SemaphoreType.DMA,
        pltpu.SemaphoreType.DMA,
    ],
)

GK = 80
_GCH = EL // GK
_GSLOT = (_GCH + 31) // 32


def _pair_gather_body(tab, sidx, didx, hs, hd, ibuf0, ibuf1, rows0, rows1,
                      gsem, wsem0, wsem1, isem0, isem1):
  c = lax.axis_index("c")
  s = lax.axis_index("s")
  wid = s * 2 + c
  ibuf = (ibuf0, ibuf1)
  rows = (rows0, rows1)
  wsem = (wsem0, wsem1)
  isem = (isem0, isem1)
  idx_in = (sidx, didx)
  out = (hs, hd)

  def issue_idx(q, cid):
    off = pl.multiple_of(cid * GK, 8)
    pltpu.async_copy(idx_in[q].at[pl.ds(off, GK)], ibuf[q], isem[q])

  issue_idx(0, wid)
  issue_idx(1, wid)

  def slot(m, carry):
    cid = m * 32 + wid

    for q in (0, 1):
      @pl.when(cid < _GCH)
      def _():
        pltpu.make_async_copy(idx_in[q].at[pl.ds(0, GK)], ibuf[q],
                              isem[q]).wait()

        @pl.when(m >= 1)
        def _():
          pltpu.make_async_copy(rows[q], out[q].at[pl.ds(0, GK), :],
                                wsem[q]).wait()
        pltpu.async_copy(tab.at[ibuf[q]], rows[q], gsem).wait()
        off = pl.multiple_of(cid * GK, 8)
        pltpu.async_copy(rows[q], out[q].at[pl.ds(off, GK), :], wsem[q])

      @pl.when(cid + 32 < _GCH)
      def _():
        issue_idx(q, cid + 32)
    return carry

  lax.fori_loop(0, _GSLOT, slot, 0)
  pltpu.make_async_copy(rows0, hs.at[pl.ds(0, GK), :], wsem0).wait()
  pltpu.make_async_copy(rows1, hd.at[pl.ds(0, GK), :], wsem1).wait()


_pair_gather = pl.kernel(
    _pair_gather_body,
    out_type=(jax.ShapeDtypeStruct((EL, H), jnp.float32),
              jax.ShapeDtypeStruct((EL, H), jnp.float32)),
    mesh=plsc.VectorSubcoreMesh(**_MESH),
    compiler_params=pltpu.CompilerParams(use_tc_tiling_on_sc=False),
    scratch_types=[
        pltpu.VMEM((GK,), jnp.int32),
        pltpu.VMEM((GK,), jnp.int32),
        pltpu.VMEM((GK, H), jnp.float32),
        pltpu.VMEM((GK, H), jnp.float32),
        pltpu.SemaphoreType.DMA,
        pltpu.SemaphoreType.DMA,
        pltpu.SemaphoreType.DMA,
        pltpu.SemaphoreType.DMA,
        pltpu.SemaphoreType.DMA,
    ],
)

BM = 1000


def _ln_tail(h, g_ref, b_ref):
  mu = jnp.mean(h, axis=1, keepdims=True)
  var = jnp.mean((h - mu) ** 2, axis=1, keepdims=True)
  return (h - mu) / jnp.sqrt(var + 1e-5) * g_ref[...] + b_ref[...]


def _b16(a):
  return a.astype(jnp.bfloat16)


def _mm(a, b):
  return jnp.dot(_b16(a), _b16(b), preferred_element_type=jnp.float32)


def _dense0_body(x_ref, agg_ref, w1_ref, b1_ref, w2_ref, b2_ref, eps_ref,
                 g_ref, b_ref, out_ref):
  eps = eps_ref[0, 0]
  u = (1.0 + eps) * x_ref[...] + agg_ref[...]
  ub = _b16(u).astype(jnp.float32)
  w1b = _b16(w1_ref[...]).astype(jnp.float32)
  pre = jnp.maximum(
      ub[:, 0:1] * w1b[0:1, :] + ub[:, 1:2] * w1b[1:2, :] + b1_ref[...],
      0.0)
  h = _mm(pre, w2_ref[...]) + b2_ref[...]
  out_ref[...] = _ln_tail(h, g_ref, b_ref)


def _dense1_body(x_ref, agg_ref, w1_ref, b1_ref, w2_ref, b2_ref, eps_ref,
                 g_ref, b_ref, out_ref):
  eps = eps_ref[0, 0]
  u = (1.0 + eps) * x_ref[...] + agg_ref[...]
  pre = jnp.maximum(_mm(u, w1_ref[...]) + b1_ref[...], 0.0)
  h = _mm(pre, w2_ref[...]) + b2_ref[...]
  out_ref[...] = _ln_tail(h, g_ref, b_ref)


def _make_dense(body, din, w1shape):
  full = lambda shp: pl.BlockSpec(shp, lambda i: (0, 0))
  return pl.pallas_call(
      body,
      grid=(N // BM,),
      in_specs=[
          pl.BlockSpec((BM, din), lambda i: (i, 0)),
          pl.BlockSpec((BM, din), lambda i: (i, 0)),
          full(w1shape),
          full((1, H)),
          full((H, H)),
          full((1, H)),
          full((1, 1)),
          full((1, H)),
          full((1, H)),
      ],
      out_specs=pl.BlockSpec((BM, H), lambda i: (i, 0)),
      out_shape=jax.ShapeDtypeStruct((N, H), jnp.float32),
  )


_dense16 = _make_dense(_dense0_body, 2, (8, H))
_dense64 = _make_dense(_dense1_body, H, (H, H))


def _head_body(hs_ref, hd_ref, wa_ref, wb_ref, b1_ref, w2_ref, b2_ref,
               w3_ref, b3_ref, out_ref):
  z = jnp.maximum(
      _mm(hs_ref[...], wa_ref[...]) + _mm(hd_ref[...], wb_ref[...])
      + b1_ref[...], 0.0)
  z = jnp.maximum(_mm(z, w2_ref[...]) + b2_ref[...], 0.0)
  zb = _b16(z).astype(jnp.float32)
  w3row = _b16(w3_ref[0:1, :]).astype(jnp.float32)
  out_ref[...] = jnp.sum(zb * w3row, axis=1, keepdims=True) + b3_ref[...]


_head = pl.pallas_call(
    _head_body,
    grid=(EL // BM,),
    in_specs=[
        pl.BlockSpec((BM, H), lambda i: (i, 0)),
        pl.BlockSpec((BM, H), lambda i: (i, 0)),
        pl.BlockSpec((H, H), lambda i: (0, 0)),
        pl.BlockSpec((H, H), lambda i: (0, 0)),
        pl.BlockSpec((1, H), lambda i: (0, 0)),
        pl.BlockSpec((H, H // 2), lambda i: (0, 0)),
        pl.BlockSpec((1, H // 2), lambda i: (0, 0)),
        pl.BlockSpec((8, H // 2), lambda i: (0, 0)),
        pl.BlockSpec((1, 1), lambda i: (0, 0)),
    ],
    out_specs=pl.BlockSpec((BM, 1), lambda i: (i, 0)),
    out_shape=jax.ShapeDtypeStruct((EL, 1), jnp.float32),
)


def kernel(x, edge_index, edge_label_index, w1, b1, w2, b2, eps0, ln0_g,
           ln0_b, w3, b3, w4, b4, eps1, ln1_g, ln1_b, we1, be1, we2, be2,
           we3, be3):
  src = jnp.concatenate([edge_index[0],
                         jnp.zeros((E_PAD - E,), jnp.int32)])
  dst = jnp.concatenate([edge_index[1],
                         jnp.full((E_PAD - E,), N, jnp.int32)])
  w1p = jnp.pad(w1, ((0, 6), (0, 0)))

  agg0_raw = _agg0(x.T, src, dst)
  agg0 = agg0_raw.reshape(2, A0_R * 16)[:, :N].T
  h0 = _dense16(x, agg0, w1p, b1.reshape(1, H), w2, b2.reshape(1, H),
                eps0.reshape(1, 1), ln0_g.reshape(1, H), ln0_b.reshape(1, H))
  agg1 = _segsum64(h0, src, dst)
  h1 = _dense64(h0, agg1, w3, b3.reshape(1, H), w4, b4.reshape(1, H),
                eps1.reshape(1, 1), ln1_g.reshape(1, H), ln1_b.reshape(1, H))
  hs, hd = _pair_gather(h1, edge_label_index[0], edge_label_index[1])
  w3p = jnp.pad(we3.T, ((0, 7), (0, 0)))
  logits = _head(hs, hd, we1[:H], we1[H:], be1.reshape(1, H), we2,
                 be2.reshape(1, H // 2), w3p, be3.reshape(1, 1))
  return logits[:, 0]
